# Initial kernel scaffold; baseline (speedup 1.0000x reference)
import jax
import jax.numpy as jnp
from jax.experimental import pallas as pl


def kernel(vertex_positions, vertex_features, img_feat0, img_feat1, img_feat2, img_feat3, vertex_adjacency, W_align, rg0_proj, rg0_c0_w0, rg0_c0_w1, rg0_c1_w0, rg0_c1_w1, rg1_c0_w0, rg1_c0_w1, rg1_c1_w0, rg1_c1_w1, rg2_c0_w0, rg2_c0_w1, rg2_c1_w0, rg2_c1_w1, gc_w0, gc_w1):
    raise NotImplementedError("write your pallas kernel here")


# speedup vs baseline: 3.8768x; 3.8768x over previous
import functools

import jax
import jax.numpy as jnp
from jax import lax
from jax.experimental import pallas as pl
from jax.experimental.pallas import tpu as pltpu
from jax.experimental.pallas import tpu_sc as plsc

NV = 10000
FD = 128
NC = 2
NS = 16
NW = NC * NS
NPAD = 10240
RPW = NPAD // NW
RPS = NPAD // NS
NE = 320000
ECHUNK = 128
EPT = 10112
EPAD = EPT * NW
BR = 512

LEVELS = ((56, 256, 3200, 0), (28, 512, 832, 256),
          (14, 1024, 256, 768), (7, 2048, 64, 1792))


def _mesh():
    return plsc.VectorSubcoreMesh(core_axis_name="c", subcore_axis_name="s")


def _align_call(xs, ys, zs, pts):
    scratch = [
        pltpu.VMEM((64,), jnp.float32),
        pltpu.VMEM((64,), jnp.float32),
        pltpu.VMEM((64,), jnp.float32),
        pltpu.VMEM((64,), jnp.int32),
        pltpu.VMEM((64,), jnp.int32),
        pltpu.VMEM((64,), jnp.int32),
        pltpu.VMEM((64,), jnp.int32),
        pltpu.VMEM((64, FD), jnp.float32),
        pltpu.VMEM((64, FD), jnp.float32),
        pltpu.VMEM((64, FD), jnp.float32),
        pltpu.VMEM((64, FD), jnp.float32),
        pltpu.VMEM((64, FD), jnp.float32),
        pltpu.SemaphoreType.DMA,
    ]

    @functools.partial(
        pl.kernel,
        out_type=jax.ShapeDtypeStruct((NPAD, FD), jnp.float32),
        mesh=_mesh(),
        scratch_types=scratch,
    )
    def body(xh, yh, zh, pt0, pt1, pt2, pt3, out,
             xv, yv, zv, i0, i1, i2, i3, g0, g1, g2, g3, ob, sem):
        cid = lax.axis_index("c")
        sid = lax.axis_index("s")
        base = (sid * NC + cid) * RPW
        tabs = (pt0, pt1, pt2, pt3)
        idxs = (i0, i1, i2, i3)
        gbufs = (g0, g1, g2, g3)

        def chunk(j, carry):
            cb = base + j * 64
            pltpu.sync_copy(xh.at[pl.ds(cb, 64)], xv)
            pltpu.sync_copy(yh.at[pl.ds(cb, 64)], yv)
            pltpu.sync_copy(zh.at[pl.ds(cb, 64)], zv)
            for i in range(4):
                sl = pl.ds(i * 16, 16)
                px, py, pz = xv[sl], yv[sl], zv[sl]
                hh = jnp.clip(248.0 * (py / pz) + 111.5, 0.0, 223.0)
                ww = jnp.clip(248.0 * (px / (-pz)) + 111.5, 0.0, 223.0)
                for (S, _, _, _), iv in zip(LEVELS, idxs):
                    sc = float(S) / 224.0
                    fx = ww * sc
                    fy = hh * sc
                    x1 = fx.astype(jnp.int32)
                    y1 = fy.astype(jnp.int32)
                    x2 = jnp.minimum(
                        jnp.where(fx > x1.astype(jnp.float32), x1 + 1, x1), S - 1)
                    y2 = jnp.minimum(
                        jnp.where(fy > y1.astype(jnp.float32), y1 + 1, y1), S - 1)
                    wb = (x2 > x1) & (y2 > y1)
                    iv[sl] = jnp.where(wb, x1 * S + y1, S * S)
            cps = [pltpu.async_copy(t.at[iv], g, sem)
                   for t, iv, g in zip(tabs, idxs, gbufs)]
            for cp in cps:
                cp.wait()

            def sumrow(r, c2):
                for c in range(FD // 16):
                    s = pl.ds(c * 16, 16)
                    ob[r, s] = g0[r, s] + g1[r, s] + g2[r, s] + g3[r, s]
                return c2
            lax.fori_loop(0, 64, sumrow, 0)
            pltpu.sync_copy(ob, out.at[pl.ds(cb, 64)])
            return carry

        lax.fori_loop(0, RPW // 64, chunk, 0)

    return body(xs, ys, zs, *pts)


def _scatter_call(f1, srcp, dstp, d):
    scratch = [
        pltpu.VMEM((64, d), jnp.float32),
        pltpu.VMEM((ECHUNK,), jnp.int32),
        pltpu.VMEM((ECHUNK,), jnp.int32),
        pltpu.VMEM((ECHUNK, d), jnp.float32),
        pltpu.VMEM_SHARED((NPAD, d), jnp.float32),
        pltpu.SemaphoreType.DMA,
    ]

    @functools.partial(
        pl.kernel,
        out_type=jax.ShapeDtypeStruct((NC, NPAD, d), jnp.float32),
        mesh=_mesh(),
        scratch_types=scratch,
    )
    def body(fh, sh, dh, out, zb, isv, idv, rows, agg, sem):
        cid = lax.axis_index("c")
        sid = lax.axis_index("s")
        ebase = (cid * NS + sid) * EPT
        zv = jnp.zeros((16,), jnp.float32)

        def zrow(r, c2):
            for c in range(d // 16):
                zb[r, pl.ds(c * 16, 16)] = zv
            return c2
        lax.fori_loop(0, 64, zrow, 0)

        def zcp(k, c2):
            pltpu.sync_copy(zb, agg.at[pl.ds(sid * RPS + k * 64, 64)])
            return c2
        lax.fori_loop(0, RPS // 64, zcp, 0)
        plsc.subcore_barrier()

        def echunk(j, c2):
            off = ebase + j * ECHUNK
            pltpu.sync_copy(sh.at[pl.ds(off, ECHUNK)], isv)
            pltpu.sync_copy(dh.at[pl.ds(off, ECHUNK)], idv)
            pltpu.async_copy(fh.at[isv], rows, sem).wait()
            pltpu.sync_copy(rows, agg.at[idv], add=True)
            return c2
        lax.fori_loop(0, EPT // ECHUNK, echunk, 0)
        plsc.subcore_barrier()
        pltpu.sync_copy(agg.at[pl.ds(sid * RPS, RPS)],
                        out.at[cid, pl.ds(sid * RPS, RPS)])

    return body(f1, srcp, dstp)


def _dot(a, b):
    return jnp.dot(a, b, preferred_element_type=jnp.float32)


def _tables_call(tins, wins):
    def body(t0, t1, t2, t3, w0, w1, w2, w3, o0, o1, o2, o3):
        for t, w, o in ((t0, w0, o0), (t1, w1, o1), (t2, w2, o2), (t3, w3, o3)):
            o[...] = lax.dot_general(
                t[...], w[...], (((0,), (0,)), ((), ())),
                preferred_element_type=jnp.float32)

    out_shape = [jax.ShapeDtypeStruct((tp, FD), jnp.float32)
                 for (_, _, tp, _) in LEVELS]
    return pl.pallas_call(body, out_shape=out_shape)(*tins, *wins)


def _first_call(vfeat, pos8, proj, w9):
    def body(vf, p8, pj, wfs, wps, wgs, wf0, wp0, wg0, wf1, wp1, wg1,
             sk, f0, f1):
        a, b, c = vf[...], p8[...], pj[...]
        for (wa, wb, wc), o in (((wfs, wps, wgs), sk),
                                ((wf0, wp0, wg0), f0),
                                ((wf1, wp1, wg1), f1)):
            o[...] = _dot(a, wa[...]) + _dot(b, wb[...]) + _dot(c, wc[...])

    row = lambda w: pl.BlockSpec((BR, w), lambda i: (i, 0))
    full = lambda s: pl.BlockSpec(s, lambda i: (0, 0))
    in_specs = [row(FD), row(8), row(FD)] + [full(w.shape) for w in w9]
    return pl.pallas_call(
        body, grid=(NPAD // BR,), in_specs=in_specs,
        out_specs=[row(FD)] * 3,
        out_shape=[jax.ShapeDtypeStruct((NPAD, FD), jnp.float32)] * 3,
    )(vfeat, pos8, proj, *w9)


def _combine_call(f0, agg, ws, skip=None, emit_h=False):
    nw = len(ws)

    def body(*refs):
        if skip is None:
            f, a = refs[:2]
            wrs = refs[2:2 + nw]
            outs = refs[2 + nw:]
            h = jnp.maximum(f[...] + a[0] + a[1], 0.0)
        else:
            f, a, sk = refs[:3]
            wrs = refs[3:3 + nw]
            outs = refs[3 + nw:]
            h = sk[...] + jnp.maximum(f[...] + a[0] + a[1], 0.0)
        i = 0
        if emit_h:
            outs[0][...] = h
            i = 1
        for j, wr in enumerate(wrs):
            outs[i + j][...] = _dot(h, wr[...])

    row = lambda w: pl.BlockSpec((BR, w), lambda i: (i, 0))
    agg_spec = pl.BlockSpec((NC, BR, FD), lambda i: (0, i, 0))
    full = lambda s: pl.BlockSpec(s, lambda i: (0, 0))
    in_specs = [row(FD), agg_spec]
    args = [f0, agg]
    if skip is not None:
        in_specs.append(row(FD))
        args.append(skip)
    in_specs += [full(w.shape) for w in ws]
    args += list(ws)
    out_specs = ([row(FD)] if emit_h else []) + [row(w.shape[1]) for w in ws]
    out_shape = (([jax.ShapeDtypeStruct((NPAD, FD), jnp.float32)] if emit_h else [])
                 + [jax.ShapeDtypeStruct((NPAD, w.shape[1]), jnp.float32)
                    for w in ws])
    return pl.pallas_call(
        body, grid=(NPAD // BR,), in_specs=in_specs,
        out_specs=out_specs, out_shape=out_shape)(*args)


def _last_call(g0, agg, w1, pos8):
    def body(g, a, w, p8, o):
        s = _dot(a[0] + a[1], w[...])
        t = jnp.tanh(jnp.maximum(g[...] + s, 0.0))
        o[...] = p8[...] + t[:, 0:8]

    row = lambda w: pl.BlockSpec((BR, w), lambda i: (i, 0))
    agg_spec = pl.BlockSpec((NC, BR, FD), lambda i: (0, i, 0))
    full = lambda s: pl.BlockSpec(s, lambda i: (0, 0))
    return pl.pallas_call(
        body, grid=(NPAD // BR,),
        in_specs=[row(16), agg_spec, full(w1.shape), row(8)],
        out_specs=row(8),
        out_shape=jax.ShapeDtypeStruct((NPAD, 8), jnp.float32),
    )(g0, agg, w1, pos8)


def kernel(vertex_positions, vertex_features, img_feat0, img_feat1,
           img_feat2, img_feat3, vertex_adjacency, W_align, rg0_proj,
           rg0_c0_w0, rg0_c0_w1, rg0_c1_w0, rg0_c1_w1, rg1_c0_w0, rg1_c0_w1,
           rg1_c1_w0, rg1_c1_w1, rg2_c0_w0, rg2_c0_w1, rg2_c1_w0, rg2_c1_w1,
           gc_w0, gc_w1):
    pos = vertex_positions
    pr = NPAD - NV
    xs = jnp.pad(pos[:, 0], (0, pr))
    ys = jnp.pad(pos[:, 1], (0, pr))
    zs = jnp.pad(pos[:, 2], (0, pr), constant_values=1.0)
    vfeat = jnp.pad(vertex_features, ((0, pr), (0, 0)))
    pos8 = jnp.pad(pos, ((0, pr), (0, 5)))

    imgs = (img_feat0, img_feat1, img_feat2, img_feat3)
    tins, wins = [], []
    for img, (S, C, TP, OFF) in zip(imgs, LEVELS):
        tins.append(jnp.pad(img[0].reshape(C, S * S), ((0, 0), (0, TP - S * S))))
        wins.append(W_align[OFF:OFF + C])

    epr = NE // NW
    src = vertex_adjacency[0].reshape(NW, epr)
    dst = vertex_adjacency[1].reshape(NW, epr)
    srcp = jnp.pad(src, ((0, 0), (0, EPT - epr))).reshape(-1)
    dstp = jnp.pad(dst, ((0, 0), (0, EPT - epr)),
                   constant_values=NV).reshape(-1)

    def split(w):
        return (w[:FD], jnp.pad(w[FD:FD + 3], ((0, 5), (0, 0))), w[FD + 3:])
    w9 = [*split(rg0_proj), *split(rg0_c0_w0), *split(rg0_c0_w1)]

    gw0 = jnp.pad(gc_w0, ((0, 0), (0, 13)))
    gw1 = jnp.pad(gc_w1, ((0, 0), (0, 13)))

    pts = _tables_call(tins, wins)
    proj = _align_call(xs, ys, zs, pts)
    sk, f0a, f1a = _first_call(vfeat, pos8, proj, w9)

    agg = _scatter_call(f1a, srcp, dstp, FD)
    f0b, f1b = _combine_call(f0a, agg, (rg0_c1_w0, rg0_c1_w1))
    agg = _scatter_call(f1b, srcp, dstp, FD)
    x0, f0c, f1c = _combine_call(f0b, agg, (rg1_c0_w0, rg1_c0_w1),
                                 skip=sk, emit_h=True)
    agg = _scatter_call(f1c, srcp, dstp, FD)
    f0d, f1d = _combine_call(f0c, agg, (rg1_c1_w0, rg1_c1_w1))
    agg = _scatter_call(f1d, srcp, dstp, FD)
    x1, f0e, f1e = _combine_call(f0d, agg, (rg2_c0_w0, rg2_c0_w1),
                                 skip=x0, emit_h=True)
    agg = _scatter_call(f1e, srcp, dstp, FD)
    f0f, f1f = _combine_call(f0e, agg, (rg2_c1_w0, rg2_c1_w1))
    agg = _scatter_call(f1f, srcp, dstp, FD)
    x2, g0 = _combine_call(f0f, agg, (gw0,), skip=x1, emit_h=True)
    agg = _scatter_call(x2, srcp, dstp, FD)
    pout = _last_call(g0, agg, gw1, pos8)

    return (pout[:NV, :3], x2[:NV])
